# baseline (speedup 1.0000x reference)
import functools

import jax
import jax.numpy as jnp
from jax import lax
from jax.experimental import pallas as pl
from jax.experimental.pallas import tpu as pltpu
from jax.experimental.pallas import tpu_sc as plsc

N = 10000
E = 320000
IN_C = 128
H_C = 128
H2_C = 64
OUT_C = 40
C1 = H_C + 8

NPAD = 10240
NC = 2
NS = 16
NW = NC * NS
EPAD = 327680
EPW = EPAD // NW
CHUNK = 128
RPS = NPAD // NS
RTC = 1280

_f32 = jnp.float32


NCH = EPW // CHUNK
G = 8
NCH0 = 80
NCH1 = 80
NG0 = NCH0 // G
NG1 = NCH1 // G


def _sc_segment_sum(pre, ed, zeros, c, stage_pre):
    mesh = plsc.VectorSubcoreMesh(core_axis_name="c", subcore_axis_name="s")

    @functools.partial(
        pl.kernel,
        out_type=jax.ShapeDtypeStruct((2 * NPAD, c), _f32),
        mesh=mesh,
        scratch_types=[
            pltpu.VMEM((G, 2, CHUNK), jnp.int32),
            pltpu.VMEM((G, 2, CHUNK), jnp.int32),
            pltpu.VMEM((CHUNK, c), _f32),
            pltpu.VMEM((CHUNK, c), _f32),
            pltpu.VMEM_SHARED((NPAD, c), _f32),
            pltpu.SemaphoreType.DMA,
            pltpu.SemaphoreType.DMA,
            pltpu.SemaphoreType.DMA,
            pltpu.SemaphoreType.DMA,
            pltpu.SemaphoreType.DMA,
            pltpu.SemaphoreType.DMA,
        ] + ([pltpu.VMEM_SHARED((NPAD, c), _f32)] if stage_pre else []),
        compiler_params=pltpu.CompilerParams(use_tc_tiling_on_sc=False),
    )
    def k(pre_hbm, ed_hbm, z_hbm, out_hbm, ga, gb,
          rows_a, rows_b, table, sia, sib, sem_a, sem_b, ssa, ssb, *rest):
        pre_sp = rest[0] if stage_pre else None
        gsrc = pre_sp if stage_pre else pre_hbm
        cid = lax.axis_index("c")
        sid = lax.axis_index("s")
        r0 = sid * RPS
        base = jnp.where(cid == 0, sid * NCH0, NS * NCH0 + sid * NCH1)
        ng_w = jnp.where(cid == 0, NG0, NG1)
        nz = RPS // CHUNK
        for z in range(nz):
            pltpu.async_copy(z_hbm.at[pl.ds(r0 + z * CHUNK, CHUNK)],
                             table.at[pl.ds(r0 + z * CHUNK, CHUNK)], sem_a)
        if stage_pre:
            for z in range(nz):
                pltpu.async_copy(pre_hbm.at[pl.ds(r0 + z * CHUNK, CHUNK)],
                                 pre_sp.at[pl.ds(r0 + z * CHUNK, CHUNK)],
                                 sem_b)
        for z in range(nz):
            pltpu.make_async_copy(z_hbm.at[pl.ds(r0, CHUNK)],
                                  table.at[pl.ds(r0, CHUNK)], sem_a).wait()
        if stage_pre:
            for z in range(nz):
                pltpu.make_async_copy(pre_hbm.at[pl.ds(r0, CHUNK)],
                                      pre_sp.at[pl.ds(r0, CHUNK)],
                                      sem_b).wait()
        plsc.subcore_barrier()

        def gload(g, buf, sem):
            pltpu.async_copy(ed_hbm.at[pl.ds(base + g * G, G)], buf, sem)

        def gload_wait(buf, sem):
            pltpu.make_async_copy(ed_hbm.at[pl.ds(0, G)], buf, sem).wait()

        def gather(buf, kk, rows, sem):
            pltpu.async_copy(gsrc.at[buf.at[kk, 0]], rows, sem)

        def gwait(rows, sem):
            pltpu.make_async_copy(gsrc.at[pl.ds(0, CHUNK)], rows, sem).wait()

        def scat(buf, kk, rows, sem):
            pltpu.async_copy(rows, table.at[buf.at[kk, 1]], sem, add=True)

        def swait(rows, sem):
            pltpu.make_async_copy(rows, table.at[pl.ds(0, CHUNK)], sem).wait()

        def run_group(g, buf, si):
            gload_wait(buf, si)
            gather(buf, 0, rows_a, sem_a)

            def body(j, carry):
                c0 = 2 * j

                @pl.when(j > 0)
                def _():
                    swait(rows_b, ssb)

                gather(buf, c0 + 1, rows_b, sem_b)
                gwait(rows_a, sem_a)
                scat(buf, c0, rows_a, ssa)
                gwait(rows_b, sem_b)
                swait(rows_a, ssa)

                @pl.when(c0 + 2 < G)
                def _():
                    gather(buf, c0 + 2, rows_a, sem_a)

                scat(buf, c0 + 1, rows_b, ssb)
                return carry

            lax.fori_loop(0, G // 2, body, 0)
            swait(rows_b, ssb)

            @pl.when(g + 2 < ng_w)
            def _():
                gload(g + 2, buf, si)

        gload(0, ga, sia)
        gload(1, gb, sib)

        def outer(g, carry):
            @pl.when(g % 2 == 0)
            def _():
                run_group(g, ga, sia)

            @pl.when(g % 2 == 1)
            def _():
                run_group(g, gb, sib)
            return carry

        lax.fori_loop(0, ng_w, outer, 0)
        plsc.subcore_barrier()
        for z in range(nz):
            pltpu.async_copy(
                table.at[pl.ds(r0 + z * CHUNK, CHUNK)],
                out_hbm.at[pl.ds(cid * NPAD + r0 + z * CHUNK, CHUNK)], sem_a)
        for z in range(nz):
            pltpu.make_async_copy(
                table.at[pl.ds(r0, CHUNK)],
                out_hbm.at[pl.ds(cid * NPAD + r0, CHUNK)], sem_a).wait()

    return k(pre, ed, zeros)


def _full(shape):
    return pl.BlockSpec(shape, lambda i: (0, 0))


def _rows(width):
    return pl.BlockSpec((RTC, width), lambda i: (i, 0))


C1A = 64
C1B = 72


def _tc_stage1(x_pad, w1l_a, w1l_b, e_row, w1r, b1_row):
    def body(x_ref, wla, wlb, er, wr, br, prea_ref, preb_ref, hr_ref):
        xb = x_ref[...]
        prea_ref[...] = jnp.dot(xb, wla[...], preferred_element_type=_f32)
        preb_ref[...] = jnp.dot(xb, wlb[...], preferred_element_type=_f32) + er[...]
        hr_ref[...] = jnp.dot(xb, wr[...], preferred_element_type=_f32) + br[...]

    return pl.pallas_call(
        body,
        grid=(NPAD // RTC,),
        in_specs=[_rows(IN_C), _full((IN_C, C1A)), _full((IN_C, C1B)),
                  _full((1, C1B)), _full((IN_C, H_C)), _full((1, H_C))],
        out_specs=[_rows(C1A), _rows(C1B), _rows(H_C)],
        out_shape=[jax.ShapeDtypeStruct((NPAD, C1A), _f32),
                   jax.ShapeDtypeStruct((NPAD, C1B), _f32),
                   jax.ShapeDtypeStruct((NPAD, H_C), _f32)],
    )(x_pad, w1l_a, w1l_b, e_row, w1r, b1_row)


def _tc_stage2(ax0, ax1, ay0, ay1, hr1, w2l, w2r, b2_row, sel):
    def body(x0, x1, y0, y1, hr, wl, wr, br, sl, pre_ref, hr_ref, rec_ref):
        sx = x0[...] + x1[...]
        sy = y0[...] + y1[...]
        cnt = jnp.dot(sy, sl[...], preferred_element_type=_f32)
        rec = 1.0 / jnp.maximum(cnt, 1.0)
        agg = jnp.concatenate([sx, sy[:, :C1A]], axis=-1)
        h1 = jnp.maximum(agg * rec + hr[...], 0.0)
        pre_ref[...] = jnp.dot(h1, wl[...], preferred_element_type=_f32)
        hr_ref[...] = jnp.dot(h1, wr[...], preferred_element_type=_f32) + br[...]
        rec_ref[...] = rec

    return pl.pallas_call(
        body,
        grid=(NPAD // RTC,),
        in_specs=[_rows(C1A), _rows(C1A), _rows(C1B), _rows(C1B),
                  _rows(H_C), _full((H_C, H2_C)),
                  _full((H_C, H2_C)), _full((1, H2_C)), _full((C1B, 1))],
        out_specs=[_rows(H2_C), _rows(H2_C), _rows(1)],
        out_shape=[jax.ShapeDtypeStruct((NPAD, H2_C), _f32),
                   jax.ShapeDtypeStruct((NPAD, H2_C), _f32),
                   jax.ShapeDtypeStruct((NPAD, 1), _f32)],
    )(ax0, ax1, ay0, ay1, hr1, w2l, w2r, b2_row, sel)


def _tc_stage3(agg_a, agg_b, hr2, rec, w3l, w3r, b3_row):
    def body(aa, ab, hr, rc, wl, wr, br, pre_ref, hr_ref):
        h2 = jnp.maximum((aa[...] + ab[...]) * rc[...] + hr[...], 0.0)
        pre_ref[...] = jnp.dot(h2, wl[...], preferred_element_type=_f32)
        hr_ref[...] = jnp.dot(h2, wr[...], preferred_element_type=_f32) + br[...]

    return pl.pallas_call(
        body,
        grid=(NPAD // RTC,),
        in_specs=[_rows(H2_C), _rows(H2_C), _rows(H2_C), _rows(1),
                  _full((H2_C, OUT_C)), _full((H2_C, OUT_C)),
                  _full((1, OUT_C))],
        out_specs=[_rows(OUT_C), _rows(OUT_C)],
        out_shape=[jax.ShapeDtypeStruct((NPAD, OUT_C), _f32),
                   jax.ShapeDtypeStruct((NPAD, OUT_C), _f32)],
    )(agg_a, agg_b, hr2, rec, w3l, w3r, b3_row)


def _tc_stage4(agg_a, agg_b, hr3, rec):
    def body(aa, ab, hr, rc, out_ref):
        z = (aa[...] + ab[...]) * rc[...] + hr[...]
        m = jnp.max(z, axis=-1, keepdims=True)
        ez = jnp.exp(z - m)
        out_ref[...] = z - m - jnp.log(jnp.sum(ez, axis=-1, keepdims=True))

    return pl.pallas_call(
        body,
        grid=(NPAD // RTC,),
        in_specs=[_rows(OUT_C), _rows(OUT_C), _rows(OUT_C), _rows(1)],
        out_specs=_rows(OUT_C),
        out_shape=jax.ShapeDtypeStruct((NPAD, OUT_C), _f32),
    )(agg_a, agg_b, hr3, rec)


def kernel(x, adj, W1l, W1r, b1, W2l, W2r, b2, W3l, W3r, b3):
    src = adj[0].astype(jnp.int32)
    dst = adj[1].astype(jnp.int32)
    npad_e = EPAD - E
    src = jnp.concatenate(
        [src, jnp.zeros((npad_e,), jnp.int32)]).reshape(EPAD // CHUNK, CHUNK)
    dst = jnp.concatenate(
        [dst, N + (jnp.arange(npad_e, dtype=jnp.int32) % (NPAD - N))]
    ).reshape(EPAD // CHUNK, CHUNK)
    ed = jnp.stack([src, dst], axis=1)

    x_pad = jnp.pad(x, ((0, NPAD - N), (0, 0)))
    w1l_a = W1l[:, :C1A]
    w1l_b = jnp.pad(W1l[:, C1A:], ((0, 0), (0, C1B - C1A)))
    e_row = jnp.zeros((1, C1B), _f32).at[0, C1A].set(1.0)
    sel = jnp.zeros((C1B, 1), _f32).at[C1A, 0].set(1.0)
    b1_row = b1.reshape(1, H_C)
    b2_row = b2.reshape(1, H2_C)
    b3_row = b3.reshape(1, OUT_C)
    z1b = jnp.zeros((NPAD, C1B), _f32)
    z2 = jnp.zeros((NPAD, H2_C), _f32)
    z3 = jnp.zeros((NPAD, OUT_C), _f32)

    pre1a, pre1b, hr1 = _tc_stage1(x_pad, w1l_a, w1l_b, e_row, W1r, b1_row)
    agg1x = _sc_segment_sum(pre1a, ed, z2, C1A, stage_pre=True)
    agg1y = _sc_segment_sum(pre1b, ed, z1b, C1B, stage_pre=True)
    pre2, hr2, rec = _tc_stage2(agg1x[:NPAD], agg1x[NPAD:],
                                agg1y[:NPAD], agg1y[NPAD:], hr1,
                                W2l, W2r, b2_row, sel)
    agg2 = _sc_segment_sum(pre2, ed, z2, H2_C, stage_pre=True)
    pre3, hr3 = _tc_stage3(agg2[:NPAD], agg2[NPAD:], hr2, rec,
                           W3l, W3r, b3_row)
    agg3 = _sc_segment_sum(pre3, ed, z3, OUT_C, stage_pre=True)
    out = _tc_stage4(agg3[:NPAD], agg3[NPAD:], hr3, rec)
    return out[:N]


# speedup vs baseline: 1.0425x; 1.0425x over previous
import functools

import jax
import jax.numpy as jnp
from jax import lax
from jax.experimental import pallas as pl
from jax.experimental.pallas import tpu as pltpu
from jax.experimental.pallas import tpu_sc as plsc

N = 10000
E = 320000
IN_C = 128
H_C = 128
H2_C = 64
OUT_C = 40
C1 = H_C + 8

NPAD = 10240
NC = 2
NS = 16
NW = NC * NS
EPAD = 327680
EPW = EPAD // NW
CHUNK = 128
RPS = NPAD // NS
RTC = 1280

_f32 = jnp.float32


NCH = EPW // CHUNK
G = 16
NCH0 = 80
NCH1 = 80
NG0 = NCH0 // G
NG1 = NCH1 // G


def _sc_segment_sum(pre, ed, zeros, c, stage_pre):
    mesh = plsc.VectorSubcoreMesh(core_axis_name="c", subcore_axis_name="s")

    @functools.partial(
        pl.kernel,
        out_type=jax.ShapeDtypeStruct((2 * NPAD, c), _f32),
        mesh=mesh,
        scratch_types=[
            pltpu.VMEM((G, 2, CHUNK), jnp.int32),
            pltpu.VMEM((G, 2, CHUNK), jnp.int32),
            pltpu.VMEM((CHUNK, c), _f32),
            pltpu.VMEM((CHUNK, c), _f32),
            pltpu.VMEM_SHARED((NPAD, c), _f32),
            pltpu.SemaphoreType.DMA,
            pltpu.SemaphoreType.DMA,
            pltpu.SemaphoreType.DMA,
            pltpu.SemaphoreType.DMA,
            pltpu.SemaphoreType.DMA,
            pltpu.SemaphoreType.DMA,
        ] + ([pltpu.VMEM_SHARED((NPAD, c), _f32)] if stage_pre else []),
        compiler_params=pltpu.CompilerParams(use_tc_tiling_on_sc=False),
    )
    def k(pre_hbm, ed_hbm, z_hbm, out_hbm, ga, gb,
          rows_a, rows_b, table, sia, sib, sem_a, sem_b, ssa, ssb, *rest):
        pre_sp = rest[0] if stage_pre else None
        gsrc = pre_sp if stage_pre else pre_hbm
        cid = lax.axis_index("c")
        sid = lax.axis_index("s")
        r0 = sid * RPS
        base = jnp.where(cid == 0, sid * NCH0, NS * NCH0 + sid * NCH1)
        ng_w = jnp.where(cid == 0, NG0, NG1)
        nz = RPS // CHUNK
        for z in range(nz):
            pltpu.async_copy(z_hbm.at[pl.ds(r0 + z * CHUNK, CHUNK)],
                             table.at[pl.ds(r0 + z * CHUNK, CHUNK)], sem_a)
        if stage_pre:
            for z in range(nz):
                pltpu.async_copy(pre_hbm.at[pl.ds(r0 + z * CHUNK, CHUNK)],
                                 pre_sp.at[pl.ds(r0 + z * CHUNK, CHUNK)],
                                 sem_b)
        for z in range(nz):
            pltpu.make_async_copy(z_hbm.at[pl.ds(r0, CHUNK)],
                                  table.at[pl.ds(r0, CHUNK)], sem_a).wait()
        if stage_pre:
            for z in range(nz):
                pltpu.make_async_copy(pre_hbm.at[pl.ds(r0, CHUNK)],
                                      pre_sp.at[pl.ds(r0, CHUNK)],
                                      sem_b).wait()
        plsc.subcore_barrier()

        def gload(g, buf, sem):
            pltpu.async_copy(ed_hbm.at[pl.ds(base + g * G, G)], buf, sem)

        def gload_wait(buf, sem):
            pltpu.make_async_copy(ed_hbm.at[pl.ds(0, G)], buf, sem).wait()

        def gather(buf, kk, rows, sem):
            pltpu.async_copy(gsrc.at[buf.at[kk, 0]], rows, sem)

        def gwait(rows, sem):
            pltpu.make_async_copy(gsrc.at[pl.ds(0, CHUNK)], rows, sem).wait()

        def scat(buf, kk, rows, sem):
            pltpu.async_copy(rows, table.at[buf.at[kk, 1]], sem, add=True)

        def swait(rows, sem):
            pltpu.make_async_copy(rows, table.at[pl.ds(0, CHUNK)], sem).wait()

        def run_group(g, buf, si):
            gload_wait(buf, si)
            gather(buf, 0, rows_a, sem_a)

            def body(j, carry):
                c0 = 2 * j

                @pl.when(j > 0)
                def _():
                    swait(rows_b, ssb)

                gather(buf, c0 + 1, rows_b, sem_b)
                gwait(rows_a, sem_a)
                scat(buf, c0, rows_a, ssa)
                gwait(rows_b, sem_b)
                swait(rows_a, ssa)

                @pl.when(c0 + 2 < G)
                def _():
                    gather(buf, c0 + 2, rows_a, sem_a)

                scat(buf, c0 + 1, rows_b, ssb)
                return carry

            lax.fori_loop(0, G // 2, body, 0)
            swait(rows_b, ssb)

            @pl.when(g + 2 < ng_w)
            def _():
                gload(g + 2, buf, si)

        gload(0, ga, sia)
        gload(1, gb, sib)

        def outer(g, carry):
            @pl.when(g % 2 == 0)
            def _():
                run_group(g, ga, sia)

            @pl.when(g % 2 == 1)
            def _():
                run_group(g, gb, sib)
            return carry

        lax.fori_loop(0, ng_w, outer, 0)
        plsc.subcore_barrier()
        for z in range(nz):
            pltpu.async_copy(
                table.at[pl.ds(r0 + z * CHUNK, CHUNK)],
                out_hbm.at[pl.ds(cid * NPAD + r0 + z * CHUNK, CHUNK)], sem_a)
        for z in range(nz):
            pltpu.make_async_copy(
                table.at[pl.ds(r0, CHUNK)],
                out_hbm.at[pl.ds(cid * NPAD + r0, CHUNK)], sem_a).wait()

    return k(pre, ed, zeros)


def _full(shape):
    return pl.BlockSpec(shape, lambda i: (0, 0))


def _rows(width):
    return pl.BlockSpec((RTC, width), lambda i: (i, 0))


C1A = 64
C1B = 72


def _tc_stage1(x_pad, w1l_a, w1l_b, e_row, w1r, b1_row):
    def body(x_ref, wla, wlb, er, wr, br, prea_ref, preb_ref, hr_ref):
        xb = x_ref[...]
        prea_ref[...] = jnp.dot(xb, wla[...], preferred_element_type=_f32)
        preb_ref[...] = jnp.dot(xb, wlb[...], preferred_element_type=_f32) + er[...]
        hr_ref[...] = jnp.dot(xb, wr[...], preferred_element_type=_f32) + br[...]

    return pl.pallas_call(
        body,
        grid=(NPAD // RTC,),
        in_specs=[_rows(IN_C), _full((IN_C, C1A)), _full((IN_C, C1B)),
                  _full((1, C1B)), _full((IN_C, H_C)), _full((1, H_C))],
        out_specs=[_rows(C1A), _rows(C1B), _rows(H_C)],
        out_shape=[jax.ShapeDtypeStruct((NPAD, C1A), _f32),
                   jax.ShapeDtypeStruct((NPAD, C1B), _f32),
                   jax.ShapeDtypeStruct((NPAD, H_C), _f32)],
    )(x_pad, w1l_a, w1l_b, e_row, w1r, b1_row)


def _tc_stage2(ax0, ax1, ay0, ay1, hr1, w2l, w2r, b2_row, sel):
    def body(x0, x1, y0, y1, hr, wl, wr, br, sl, pre_ref, hr_ref, rec_ref):
        sx = x0[...] + x1[...]
        sy = y0[...] + y1[...]
        cnt = jnp.dot(sy, sl[...], preferred_element_type=_f32)
        rec = 1.0 / jnp.maximum(cnt, 1.0)
        agg = jnp.concatenate([sx, sy[:, :C1A]], axis=-1)
        h1 = jnp.maximum(agg * rec + hr[...], 0.0)
        pre_ref[...] = jnp.dot(h1, wl[...], preferred_element_type=_f32)
        hr_ref[...] = jnp.dot(h1, wr[...], preferred_element_type=_f32) + br[...]
        rec_ref[...] = rec

    return pl.pallas_call(
        body,
        grid=(NPAD // RTC,),
        in_specs=[_rows(C1A), _rows(C1A), _rows(C1B), _rows(C1B),
                  _rows(H_C), _full((H_C, H2_C)),
                  _full((H_C, H2_C)), _full((1, H2_C)), _full((C1B, 1))],
        out_specs=[_rows(H2_C), _rows(H2_C), _rows(1)],
        out_shape=[jax.ShapeDtypeStruct((NPAD, H2_C), _f32),
                   jax.ShapeDtypeStruct((NPAD, H2_C), _f32),
                   jax.ShapeDtypeStruct((NPAD, 1), _f32)],
    )(ax0, ax1, ay0, ay1, hr1, w2l, w2r, b2_row, sel)


def _tc_stage3(agg_a, agg_b, hr2, rec, w3l, w3r, b3_row):
    def body(aa, ab, hr, rc, wl, wr, br, pre_ref, hr_ref):
        h2 = jnp.maximum((aa[...] + ab[...]) * rc[...] + hr[...], 0.0)
        pre_ref[...] = jnp.dot(h2, wl[...], preferred_element_type=_f32)
        hr_ref[...] = jnp.dot(h2, wr[...], preferred_element_type=_f32) + br[...]

    return pl.pallas_call(
        body,
        grid=(NPAD // RTC,),
        in_specs=[_rows(H2_C), _rows(H2_C), _rows(H2_C), _rows(1),
                  _full((H2_C, OUT_C)), _full((H2_C, OUT_C)),
                  _full((1, OUT_C))],
        out_specs=[_rows(OUT_C), _rows(OUT_C)],
        out_shape=[jax.ShapeDtypeStruct((NPAD, OUT_C), _f32),
                   jax.ShapeDtypeStruct((NPAD, OUT_C), _f32)],
    )(agg_a, agg_b, hr2, rec, w3l, w3r, b3_row)


def _tc_stage4(agg_a, agg_b, hr3, rec):
    def body(aa, ab, hr, rc, out_ref):
        z = (aa[...] + ab[...]) * rc[...] + hr[...]
        m = jnp.max(z, axis=-1, keepdims=True)
        ez = jnp.exp(z - m)
        out_ref[...] = z - m - jnp.log(jnp.sum(ez, axis=-1, keepdims=True))

    return pl.pallas_call(
        body,
        grid=(NPAD // RTC,),
        in_specs=[_rows(OUT_C), _rows(OUT_C), _rows(OUT_C), _rows(1)],
        out_specs=_rows(OUT_C),
        out_shape=jax.ShapeDtypeStruct((NPAD, OUT_C), _f32),
    )(agg_a, agg_b, hr3, rec)


def kernel(x, adj, W1l, W1r, b1, W2l, W2r, b2, W3l, W3r, b3):
    src = adj[0].astype(jnp.int32)
    dst = adj[1].astype(jnp.int32)
    npad_e = EPAD - E
    src = jnp.concatenate(
        [src, jnp.zeros((npad_e,), jnp.int32)]).reshape(EPAD // CHUNK, CHUNK)
    dst = jnp.concatenate(
        [dst, N + (jnp.arange(npad_e, dtype=jnp.int32) % (NPAD - N))]
    ).reshape(EPAD // CHUNK, CHUNK)
    ed = jnp.stack([src, dst], axis=1)

    x_pad = jnp.pad(x, ((0, NPAD - N), (0, 0)))
    w1l_a = W1l[:, :C1A]
    w1l_b = jnp.pad(W1l[:, C1A:], ((0, 0), (0, C1B - C1A)))
    e_row = jnp.zeros((1, C1B), _f32).at[0, C1A].set(1.0)
    sel = jnp.zeros((C1B, 1), _f32).at[C1A, 0].set(1.0)
    b1_row = b1.reshape(1, H_C)
    b2_row = b2.reshape(1, H2_C)
    b3_row = b3.reshape(1, OUT_C)
    z1b = jnp.zeros((NPAD, C1B), _f32)
    z2 = jnp.zeros((NPAD, H2_C), _f32)
    z3 = jnp.zeros((NPAD, OUT_C), _f32)

    pre1a, pre1b, hr1 = _tc_stage1(x_pad, w1l_a, w1l_b, e_row, W1r, b1_row)
    agg1x = _sc_segment_sum(pre1a, ed, z2, C1A, stage_pre=True)
    agg1y = _sc_segment_sum(pre1b, ed, z1b, C1B, stage_pre=True)
    pre2, hr2, rec = _tc_stage2(agg1x[:NPAD], agg1x[NPAD:],
                                agg1y[:NPAD], agg1y[NPAD:], hr1,
                                W2l, W2r, b2_row, sel)
    agg2 = _sc_segment_sum(pre2, ed, z2, H2_C, stage_pre=True)
    pre3, hr3 = _tc_stage3(agg2[:NPAD], agg2[NPAD:], hr2, rec,
                           W3l, W3r, b3_row)
    agg3 = _sc_segment_sum(pre3, ed, z3, OUT_C, stage_pre=True)
    out = _tc_stage4(agg3[:NPAD], agg3[NPAD:], hr3, rec)
    return out[:N]


# speedup vs baseline: 1.0494x; 1.0066x over previous
import functools

import jax
import jax.numpy as jnp
from jax import lax
from jax.experimental import pallas as pl
from jax.experimental.pallas import tpu as pltpu
from jax.experimental.pallas import tpu_sc as plsc

N = 10000
E = 320000
IN_C = 128
H_C = 128
H2_C = 64
OUT_C = 40
C1 = H_C + 8

NPAD = 10240
NC = 2
NS = 16
NW = NC * NS
EPAD = 327680
EPW = EPAD // NW
CHUNK = 128
RPS = NPAD // NS
RTC = 1280

_f32 = jnp.float32


NCH = EPW // CHUNK
G = 20
NCH0 = 80
NCH1 = 80
NG0 = NCH0 // G
NG1 = NCH1 // G


def _sc_segment_sum(pre, ed, zeros, c, stage_pre):
    mesh = plsc.VectorSubcoreMesh(core_axis_name="c", subcore_axis_name="s")

    @functools.partial(
        pl.kernel,
        out_type=jax.ShapeDtypeStruct((2 * NPAD, c), _f32),
        mesh=mesh,
        scratch_types=[
            pltpu.VMEM((G, 2, CHUNK), jnp.int32),
            pltpu.VMEM((G, 2, CHUNK), jnp.int32),
            pltpu.VMEM((CHUNK, c), _f32),
            pltpu.VMEM((CHUNK, c), _f32),
            pltpu.VMEM_SHARED((NPAD, c), _f32),
            pltpu.SemaphoreType.DMA,
            pltpu.SemaphoreType.DMA,
            pltpu.SemaphoreType.DMA,
            pltpu.SemaphoreType.DMA,
            pltpu.SemaphoreType.DMA,
            pltpu.SemaphoreType.DMA,
        ] + ([pltpu.VMEM_SHARED((NPAD, c), _f32)] if stage_pre else []),
        compiler_params=pltpu.CompilerParams(use_tc_tiling_on_sc=False),
    )
    def k(pre_hbm, ed_hbm, z_hbm, out_hbm, ga, gb,
          rows_a, rows_b, table, sia, sib, sem_a, sem_b, ssa, ssb, *rest):
        pre_sp = rest[0] if stage_pre else None
        gsrc = pre_sp if stage_pre else pre_hbm
        cid = lax.axis_index("c")
        sid = lax.axis_index("s")
        r0 = sid * RPS
        base = jnp.where(cid == 0, sid * NCH0, NS * NCH0 + sid * NCH1)
        ng_w = jnp.where(cid == 0, NG0, NG1)
        nz = RPS // CHUNK
        for z in range(nz):
            pltpu.async_copy(z_hbm.at[pl.ds(r0 + z * CHUNK, CHUNK)],
                             table.at[pl.ds(r0 + z * CHUNK, CHUNK)], sem_a)
        if stage_pre:
            for z in range(nz):
                pltpu.async_copy(pre_hbm.at[pl.ds(r0 + z * CHUNK, CHUNK)],
                                 pre_sp.at[pl.ds(r0 + z * CHUNK, CHUNK)],
                                 sem_b)
        for z in range(nz):
            pltpu.make_async_copy(z_hbm.at[pl.ds(r0, CHUNK)],
                                  table.at[pl.ds(r0, CHUNK)], sem_a).wait()
        if stage_pre:
            for z in range(nz):
                pltpu.make_async_copy(pre_hbm.at[pl.ds(r0, CHUNK)],
                                      pre_sp.at[pl.ds(r0, CHUNK)],
                                      sem_b).wait()
        plsc.subcore_barrier()

        def gload(g, buf, sem):
            pltpu.async_copy(ed_hbm.at[pl.ds(base + g * G, G)], buf, sem)

        def gload_wait(buf, sem):
            pltpu.make_async_copy(ed_hbm.at[pl.ds(0, G)], buf, sem).wait()

        def gather(buf, kk, rows, sem):
            pltpu.async_copy(gsrc.at[buf.at[kk, 0]], rows, sem)

        def gwait(rows, sem):
            pltpu.make_async_copy(gsrc.at[pl.ds(0, CHUNK)], rows, sem).wait()

        def scat(buf, kk, rows, sem):
            pltpu.async_copy(rows, table.at[buf.at[kk, 1]], sem, add=True)

        def swait(rows, sem):
            pltpu.make_async_copy(rows, table.at[pl.ds(0, CHUNK)], sem).wait()

        def run_group(g, buf, si):
            gload_wait(buf, si)
            gather(buf, 0, rows_a, sem_a)

            def body(j, carry):
                c0 = 2 * j

                @pl.when(j > 0)
                def _():
                    swait(rows_b, ssb)

                gather(buf, c0 + 1, rows_b, sem_b)
                gwait(rows_a, sem_a)
                scat(buf, c0, rows_a, ssa)
                gwait(rows_b, sem_b)
                swait(rows_a, ssa)

                @pl.when(c0 + 2 < G)
                def _():
                    gather(buf, c0 + 2, rows_a, sem_a)

                scat(buf, c0 + 1, rows_b, ssb)
                return carry

            lax.fori_loop(0, G // 2, body, 0)
            swait(rows_b, ssb)

            @pl.when(g + 2 < ng_w)
            def _():
                gload(g + 2, buf, si)

        gload(0, ga, sia)
        gload(1, gb, sib)

        def outer(g, carry):
            @pl.when(g % 2 == 0)
            def _():
                run_group(g, ga, sia)

            @pl.when(g % 2 == 1)
            def _():
                run_group(g, gb, sib)
            return carry

        lax.fori_loop(0, ng_w, outer, 0)
        plsc.subcore_barrier()
        for z in range(nz):
            pltpu.async_copy(
                table.at[pl.ds(r0 + z * CHUNK, CHUNK)],
                out_hbm.at[pl.ds(cid * NPAD + r0 + z * CHUNK, CHUNK)], sem_a)
        for z in range(nz):
            pltpu.make_async_copy(
                table.at[pl.ds(r0, CHUNK)],
                out_hbm.at[pl.ds(cid * NPAD + r0, CHUNK)], sem_a).wait()

    return k(pre, ed, zeros)


def _full(shape):
    return pl.BlockSpec(shape, lambda i: (0, 0))


def _rows(width):
    return pl.BlockSpec((RTC, width), lambda i: (i, 0))


C1A = 64
C1B = 72


def _tc_stage1(x_pad, w1l_a, w1l_b, e_row, w1r, b1_row):
    def body(x_ref, wla, wlb, er, wr, br, prea_ref, preb_ref, hr_ref):
        xb = x_ref[...]
        prea_ref[...] = jnp.dot(xb, wla[...], preferred_element_type=_f32)
        preb_ref[...] = jnp.dot(xb, wlb[...], preferred_element_type=_f32) + er[...]
        hr_ref[...] = jnp.dot(xb, wr[...], preferred_element_type=_f32) + br[...]

    return pl.pallas_call(
        body,
        grid=(NPAD // RTC,),
        in_specs=[_rows(IN_C), _full((IN_C, C1A)), _full((IN_C, C1B)),
                  _full((1, C1B)), _full((IN_C, H_C)), _full((1, H_C))],
        out_specs=[_rows(C1A), _rows(C1B), _rows(H_C)],
        out_shape=[jax.ShapeDtypeStruct((NPAD, C1A), _f32),
                   jax.ShapeDtypeStruct((NPAD, C1B), _f32),
                   jax.ShapeDtypeStruct((NPAD, H_C), _f32)],
    )(x_pad, w1l_a, w1l_b, e_row, w1r, b1_row)


def _tc_stage2(ax0, ax1, ay0, ay1, hr1, w2l, w2r, b2_row, sel):
    def body(x0, x1, y0, y1, hr, wl, wr, br, sl, pre_ref, hr_ref, rec_ref):
        sx = x0[...] + x1[...]
        sy = y0[...] + y1[...]
        cnt = jnp.dot(sy, sl[...], preferred_element_type=_f32)
        rec = 1.0 / jnp.maximum(cnt, 1.0)
        agg = jnp.concatenate([sx, sy[:, :C1A]], axis=-1)
        h1 = jnp.maximum(agg * rec + hr[...], 0.0)
        pre_ref[...] = jnp.dot(h1, wl[...], preferred_element_type=_f32)
        hr_ref[...] = jnp.dot(h1, wr[...], preferred_element_type=_f32) + br[...]
        rec_ref[...] = rec

    return pl.pallas_call(
        body,
        grid=(NPAD // RTC,),
        in_specs=[_rows(C1A), _rows(C1A), _rows(C1B), _rows(C1B),
                  _rows(H_C), _full((H_C, H2_C)),
                  _full((H_C, H2_C)), _full((1, H2_C)), _full((C1B, 1))],
        out_specs=[_rows(H2_C), _rows(H2_C), _rows(1)],
        out_shape=[jax.ShapeDtypeStruct((NPAD, H2_C), _f32),
                   jax.ShapeDtypeStruct((NPAD, H2_C), _f32),
                   jax.ShapeDtypeStruct((NPAD, 1), _f32)],
    )(ax0, ax1, ay0, ay1, hr1, w2l, w2r, b2_row, sel)


def _tc_stage3(agg_a, agg_b, hr2, rec, w3l, w3r, b3_row):
    def body(aa, ab, hr, rc, wl, wr, br, pre_ref, hr_ref):
        h2 = jnp.maximum((aa[...] + ab[...]) * rc[...] + hr[...], 0.0)
        pre_ref[...] = jnp.dot(h2, wl[...], preferred_element_type=_f32)
        hr_ref[...] = jnp.dot(h2, wr[...], preferred_element_type=_f32) + br[...]

    return pl.pallas_call(
        body,
        grid=(NPAD // RTC,),
        in_specs=[_rows(H2_C), _rows(H2_C), _rows(H2_C), _rows(1),
                  _full((H2_C, OUT_C)), _full((H2_C, OUT_C)),
                  _full((1, OUT_C))],
        out_specs=[_rows(OUT_C), _rows(OUT_C)],
        out_shape=[jax.ShapeDtypeStruct((NPAD, OUT_C), _f32),
                   jax.ShapeDtypeStruct((NPAD, OUT_C), _f32)],
    )(agg_a, agg_b, hr2, rec, w3l, w3r, b3_row)


def _tc_stage4(agg_a, agg_b, hr3, rec):
    def body(aa, ab, hr, rc, out_ref):
        z = (aa[...] + ab[...]) * rc[...] + hr[...]
        m = jnp.max(z, axis=-1, keepdims=True)
        ez = jnp.exp(z - m)
        out_ref[...] = z - m - jnp.log(jnp.sum(ez, axis=-1, keepdims=True))

    return pl.pallas_call(
        body,
        grid=(NPAD // RTC,),
        in_specs=[_rows(OUT_C), _rows(OUT_C), _rows(OUT_C), _rows(1)],
        out_specs=_rows(OUT_C),
        out_shape=jax.ShapeDtypeStruct((NPAD, OUT_C), _f32),
    )(agg_a, agg_b, hr3, rec)


def kernel(x, adj, W1l, W1r, b1, W2l, W2r, b2, W3l, W3r, b3):
    src = adj[0].astype(jnp.int32)
    dst = adj[1].astype(jnp.int32)
    npad_e = EPAD - E
    src = jnp.concatenate(
        [src, jnp.zeros((npad_e,), jnp.int32)]).reshape(EPAD // CHUNK, CHUNK)
    dst = jnp.concatenate(
        [dst, N + (jnp.arange(npad_e, dtype=jnp.int32) % (NPAD - N))]
    ).reshape(EPAD // CHUNK, CHUNK)
    ed = jnp.stack([src, dst], axis=1)

    x_pad = jnp.pad(x, ((0, NPAD - N), (0, 0)))
    w1l_a = W1l[:, :C1A]
    w1l_b = jnp.pad(W1l[:, C1A:], ((0, 0), (0, C1B - C1A)))
    e_row = jnp.zeros((1, C1B), _f32).at[0, C1A].set(1.0)
    sel = jnp.zeros((C1B, 1), _f32).at[C1A, 0].set(1.0)
    b1_row = b1.reshape(1, H_C)
    b2_row = b2.reshape(1, H2_C)
    b3_row = b3.reshape(1, OUT_C)
    z1b = jnp.zeros((NPAD, C1B), _f32)
    z2 = jnp.zeros((NPAD, H2_C), _f32)
    z3 = jnp.zeros((NPAD, OUT_C), _f32)

    pre1a, pre1b, hr1 = _tc_stage1(x_pad, w1l_a, w1l_b, e_row, W1r, b1_row)
    agg1x = _sc_segment_sum(pre1a, ed, z2, C1A, stage_pre=True)
    agg1y = _sc_segment_sum(pre1b, ed, z1b, C1B, stage_pre=True)
    pre2, hr2, rec = _tc_stage2(agg1x[:NPAD], agg1x[NPAD:],
                                agg1y[:NPAD], agg1y[NPAD:], hr1,
                                W2l, W2r, b2_row, sel)
    agg2 = _sc_segment_sum(pre2, ed, z2, H2_C, stage_pre=True)
    pre3, hr3 = _tc_stage3(agg2[:NPAD], agg2[NPAD:], hr2, rec,
                           W3l, W3r, b3_row)
    agg3 = _sc_segment_sum(pre3, ed, z3, OUT_C, stage_pre=True)
    out = _tc_stage4(agg3[:NPAD], agg3[NPAD:], hr3, rec)
    return out[:N]


# speedup vs baseline: 1.0514x; 1.0019x over previous
import functools

import jax
import jax.numpy as jnp
from jax import lax
from jax.experimental import pallas as pl
from jax.experimental.pallas import tpu as pltpu
from jax.experimental.pallas import tpu_sc as plsc

N = 10000
E = 320000
IN_C = 128
H_C = 128
H2_C = 64
OUT_C = 40

NPAD = 10240
NC = 2
NS = 16
NW = NC * NS
EPAD = 327680
EPW = EPAD // NW
CHUNK = 128
RPS = NPAD // NS
RTC = 1280

_f32 = jnp.float32


NCH = EPW // CHUNK
G = 20
NCH0 = 80
NCH1 = 80
NG0 = NCH0 // G
NG1 = NCH1 // G


def _sc_segment_sum(pre, ed, zeros, c, stage_pre):
    mesh = plsc.VectorSubcoreMesh(core_axis_name="c", subcore_axis_name="s")

    @functools.partial(
        pl.kernel,
        out_type=jax.ShapeDtypeStruct((2 * NPAD, c), _f32),
        mesh=mesh,
        scratch_types=[
            pltpu.VMEM((G, 2, CHUNK), jnp.int32),
            pltpu.VMEM((G, 2, CHUNK), jnp.int32),
            pltpu.VMEM((CHUNK, c), _f32),
            pltpu.VMEM((CHUNK, c), _f32),
            pltpu.VMEM_SHARED((NPAD, c), _f32),
            pltpu.SemaphoreType.DMA,
            pltpu.SemaphoreType.DMA,
            pltpu.SemaphoreType.DMA,
            pltpu.SemaphoreType.DMA,
            pltpu.SemaphoreType.DMA,
            pltpu.SemaphoreType.DMA,
        ] + ([pltpu.VMEM_SHARED((NPAD, c), _f32)] if stage_pre else []),
        compiler_params=pltpu.CompilerParams(use_tc_tiling_on_sc=False),
    )
    def k(pre_hbm, ed_hbm, z_hbm, out_hbm, ga, gb,
          rows_a, rows_b, table, sia, sib, sem_a, sem_b, ssa, ssb, *rest):
        pre_sp = rest[0] if stage_pre else None
        gsrc = pre_sp if stage_pre else pre_hbm
        cid = lax.axis_index("c")
        sid = lax.axis_index("s")
        r0 = sid * RPS
        base = jnp.where(cid == 0, sid * NCH0, NS * NCH0 + sid * NCH1)
        ng_w = jnp.where(cid == 0, NG0, NG1)
        nz = RPS // CHUNK
        for z in range(nz):
            pltpu.async_copy(z_hbm.at[pl.ds(r0 + z * CHUNK, CHUNK)],
                             table.at[pl.ds(r0 + z * CHUNK, CHUNK)], sem_a)
        if stage_pre:
            for z in range(nz):
                pltpu.async_copy(pre_hbm.at[pl.ds(r0 + z * CHUNK, CHUNK)],
                                 pre_sp.at[pl.ds(r0 + z * CHUNK, CHUNK)],
                                 sem_b)
        for z in range(nz):
            pltpu.make_async_copy(z_hbm.at[pl.ds(r0, CHUNK)],
                                  table.at[pl.ds(r0, CHUNK)], sem_a).wait()
        if stage_pre:
            for z in range(nz):
                pltpu.make_async_copy(pre_hbm.at[pl.ds(r0, CHUNK)],
                                      pre_sp.at[pl.ds(r0, CHUNK)],
                                      sem_b).wait()
        plsc.subcore_barrier()

        def gload(g, buf, sem):
            pltpu.async_copy(ed_hbm.at[pl.ds(base + g * G, G)], buf, sem)

        def gload_wait(buf, sem):
            pltpu.make_async_copy(ed_hbm.at[pl.ds(0, G)], buf, sem).wait()

        def gather(buf, kk, rows, sem):
            pltpu.async_copy(gsrc.at[buf.at[kk, 0]], rows, sem)

        def gwait(rows, sem):
            pltpu.make_async_copy(gsrc.at[pl.ds(0, CHUNK)], rows, sem).wait()

        def scat(buf, kk, rows, sem):
            pltpu.async_copy(rows, table.at[buf.at[kk, 1]], sem, add=True)

        def swait(rows, sem):
            pltpu.make_async_copy(rows, table.at[pl.ds(0, CHUNK)], sem).wait()

        def run_group(g, buf, si):
            gload_wait(buf, si)
            gather(buf, 0, rows_a, sem_a)

            def body(j, carry):
                c0 = 2 * j

                @pl.when(j > 0)
                def _():
                    swait(rows_b, ssb)

                gather(buf, c0 + 1, rows_b, sem_b)
                gwait(rows_a, sem_a)
                scat(buf, c0, rows_a, ssa)
                gwait(rows_b, sem_b)
                swait(rows_a, ssa)

                @pl.when(c0 + 2 < G)
                def _():
                    gather(buf, c0 + 2, rows_a, sem_a)

                scat(buf, c0 + 1, rows_b, ssb)
                return carry

            lax.fori_loop(0, G // 2, body, 0)
            swait(rows_b, ssb)

            @pl.when(g + 2 < ng_w)
            def _():
                gload(g + 2, buf, si)

        gload(0, ga, sia)
        gload(1, gb, sib)

        def outer(g, carry):
            @pl.when(g % 2 == 0)
            def _():
                run_group(g, ga, sia)

            @pl.when(g % 2 == 1)
            def _():
                run_group(g, gb, sib)
            return carry

        lax.fori_loop(0, ng_w, outer, 0)
        plsc.subcore_barrier()
        for z in range(nz):
            pltpu.async_copy(
                table.at[pl.ds(r0 + z * CHUNK, CHUNK)],
                out_hbm.at[pl.ds(cid * NPAD + r0 + z * CHUNK, CHUNK)], sem_a)
        for z in range(nz):
            pltpu.make_async_copy(
                table.at[pl.ds(r0, CHUNK)],
                out_hbm.at[pl.ds(cid * NPAD + r0, CHUNK)], sem_a).wait()

    return k(pre, ed, zeros)


def _full(shape):
    return pl.BlockSpec(shape, lambda i: (0, 0))


def _rows(width):
    return pl.BlockSpec((RTC, width), lambda i: (i, 0))


C1A = 64
C1B = 72


def _tc_stage1(x_pad, w1l_a, w1l_b, e_row, w1r, b1_row):
    def body(x_ref, wla, wlb, er, wr, br, prea_ref, preb_ref, hr_ref):
        xb = x_ref[...]
        prea_ref[...] = jnp.dot(xb, wla[...], preferred_element_type=_f32)
        preb_ref[...] = jnp.dot(xb, wlb[...], preferred_element_type=_f32) + er[...]
        hr_ref[...] = jnp.dot(xb, wr[...], preferred_element_type=_f32) + br[...]

    return pl.pallas_call(
        body,
        grid=(NPAD // RTC,),
        in_specs=[_rows(IN_C), _full((IN_C, C1A)), _full((IN_C, C1B)),
                  _full((1, C1B)), _full((IN_C, H_C)), _full((1, H_C))],
        out_specs=[_rows(C1A), _rows(C1B), _rows(H_C)],
        out_shape=[jax.ShapeDtypeStruct((NPAD, C1A), _f32),
                   jax.ShapeDtypeStruct((NPAD, C1B), _f32),
                   jax.ShapeDtypeStruct((NPAD, H_C), _f32)],
    )(x_pad, w1l_a, w1l_b, e_row, w1r, b1_row)


def _tc_stage2(ax0, ax1, ay0, ay1, hr1, w2l, w2r, b2_row, sel):
    def body(x0, x1, y0, y1, hr, wl, wr, br, sl, pre_ref, hr_ref, rec_ref):
        sx = x0[...] + x1[...]
        sy = y0[...] + y1[...]
        cnt = jnp.dot(sy, sl[...], preferred_element_type=_f32)
        rec = 1.0 / jnp.maximum(cnt, 1.0)
        agg = jnp.concatenate([sx, sy[:, :C1A]], axis=-1)
        h1 = jnp.maximum(agg * rec + hr[...], 0.0)
        pre_ref[...] = jnp.dot(h1, wl[...], preferred_element_type=_f32)
        hr_ref[...] = jnp.dot(h1, wr[...], preferred_element_type=_f32) + br[...]
        rec_ref[...] = rec

    return pl.pallas_call(
        body,
        grid=(NPAD // RTC,),
        in_specs=[_rows(C1A), _rows(C1A), _rows(C1B), _rows(C1B),
                  _rows(H_C), _full((H_C, H2_C)),
                  _full((H_C, H2_C)), _full((1, H2_C)), _full((C1B, 1))],
        out_specs=[_rows(H2_C), _rows(H2_C), _rows(1)],
        out_shape=[jax.ShapeDtypeStruct((NPAD, H2_C), _f32),
                   jax.ShapeDtypeStruct((NPAD, H2_C), _f32),
                   jax.ShapeDtypeStruct((NPAD, 1), _f32)],
    )(ax0, ax1, ay0, ay1, hr1, w2l, w2r, b2_row, sel)


def _tc_stage3(agg_a, agg_b, hr2, rec, w3l, w3r, b3_row):
    def body(aa, ab, hr, rc, wl, wr, br, pre_ref, hr_ref):
        h2 = jnp.maximum((aa[...] + ab[...]) * rc[...] + hr[...], 0.0)
        pre_ref[...] = jnp.dot(h2, wl[...], preferred_element_type=_f32)
        hr_ref[...] = jnp.dot(h2, wr[...], preferred_element_type=_f32) + br[...]

    return pl.pallas_call(
        body,
        grid=(NPAD // RTC,),
        in_specs=[_rows(H2_C), _rows(H2_C), _rows(H2_C), _rows(1),
                  _full((H2_C, OUT_C)), _full((H2_C, OUT_C)),
                  _full((1, OUT_C))],
        out_specs=[_rows(OUT_C), _rows(OUT_C)],
        out_shape=[jax.ShapeDtypeStruct((NPAD, OUT_C), _f32),
                   jax.ShapeDtypeStruct((NPAD, OUT_C), _f32)],
    )(agg_a, agg_b, hr2, rec, w3l, w3r, b3_row)


def _tc_stage4(agg_a, agg_b, hr3, rec):
    def body(aa, ab, hr, rc, out_ref):
        z = (aa[...] + ab[...]) * rc[...] + hr[...]
        m = jnp.max(z, axis=-1, keepdims=True)
        ez = jnp.exp(z - m)
        out_ref[...] = z - m - jnp.log(jnp.sum(ez, axis=-1, keepdims=True))

    return pl.pallas_call(
        body,
        grid=(NPAD // RTC,),
        in_specs=[_rows(OUT_C), _rows(OUT_C), _rows(OUT_C), _rows(1)],
        out_specs=_rows(OUT_C),
        out_shape=jax.ShapeDtypeStruct((NPAD, OUT_C), _f32),
    )(agg_a, agg_b, hr3, rec)


def kernel(x, adj, W1l, W1r, b1, W2l, W2r, b2, W3l, W3r, b3):
    src = adj[0].astype(jnp.int32)
    dst = adj[1].astype(jnp.int32)
    npad_e = EPAD - E
    src = jnp.concatenate(
        [src, jnp.zeros((npad_e,), jnp.int32)]).reshape(EPAD // CHUNK, CHUNK)
    dst = jnp.concatenate(
        [dst, N + (jnp.arange(npad_e, dtype=jnp.int32) % (NPAD - N))]
    ).reshape(EPAD // CHUNK, CHUNK)
    ed = jnp.stack([src, dst], axis=1)

    x_pad = jnp.pad(x, ((0, NPAD - N), (0, 0)))
    w1l_a = W1l[:, :C1A]
    w1l_b = jnp.pad(W1l[:, C1A:], ((0, 0), (0, C1B - C1A)))
    e_row = jnp.zeros((1, C1B), _f32).at[0, C1A].set(1.0)
    sel = jnp.zeros((C1B, 1), _f32).at[C1A, 0].set(1.0)
    b1_row = b1.reshape(1, H_C)
    b2_row = b2.reshape(1, H2_C)
    b3_row = b3.reshape(1, OUT_C)
    z1b = jnp.zeros((NPAD, C1B), _f32)
    z2 = jnp.zeros((NPAD, H2_C), _f32)
    z3 = jnp.zeros((NPAD, OUT_C), _f32)

    pre1a, pre1b, hr1 = _tc_stage1(x_pad, w1l_a, w1l_b, e_row, W1r, b1_row)
    agg1x = _sc_segment_sum(pre1a, ed, z2, C1A, stage_pre=True)
    agg1y = _sc_segment_sum(pre1b, ed, z1b, C1B, stage_pre=True)
    pre2, hr2, rec = _tc_stage2(agg1x[:NPAD], agg1x[NPAD:],
                                agg1y[:NPAD], agg1y[NPAD:], hr1,
                                W2l, W2r, b2_row, sel)
    agg2 = _sc_segment_sum(pre2, ed, z2, H2_C, stage_pre=True)
    pre3, hr3 = _tc_stage3(agg2[:NPAD], agg2[NPAD:], hr2, rec,
                           W3l, W3r, b3_row)
    agg3 = _sc_segment_sum(pre3, ed, z3, OUT_C, stage_pre=True)
    out = _tc_stage4(agg3[:NPAD], agg3[NPAD:], hr3, rec)
    return out[:N]
